# baseline (speedup 1.0000x reference)
import jax
import jax.numpy as jnp
from jax import lax
from jax.experimental import pallas as pl
from jax.experimental.pallas import tpu as pltpu
from jax.experimental.pallas import tpu_sc as plsc

N = 10000
E = 320000
IN_CH = 128
HID = 64
HEADS = 4

NC = 2
NS = 16
NW = NC * NS
L = 16

EB = 80
EPT = E // NS
NBATCH = EPT // EB
EPW = E // NW
NP = 10240
ROWS_PT = NP // NS
ZR = 32
DW = 16

BR = 200
GRID = N // BR

_SC_PARAMS = pltpu.CompilerParams(needs_layout_passes=False)


def _sc_mesh():
  return plsc.VectorSubcoreMesh(core_axis_name="c", subcore_axis_name="s")


def _nm_body(src_hbm, out_hbm, nm_v, idx_v):
  c = lax.axis_index("c")
  s = lax.axis_index("s")
  w = s * NC + c
  ones = jnp.ones((L,), jnp.float32)

  def zero(i, carry):
    nm_v[pl.ds(i * L, L)] = jnp.zeros((L,), jnp.float32)
    return carry
  lax.fori_loop(0, N // L, zero, 0)

  pltpu.sync_copy(src_hbm.at[pl.ds(w * EPW, EPW)], idx_v)

  def scat(i, carry):
    iv = idx_v[pl.ds(i * L, L)]
    plsc.store_scatter(nm_v, [iv], ones)
    return carry
  lax.fori_loop(0, EPW // L, scat, 0)

  pltpu.sync_copy(nm_v, out_hbm.at[pl.ds(w * N, N)])


def _nm_call(src):
  return pl.kernel(
      _nm_body,
      out_type=jax.ShapeDtypeStruct((NW * N,), jnp.float32),
      mesh=_sc_mesh(),
      compiler_params=_SC_PARAMS,
      scratch_types=[
          pltpu.VMEM((N,), jnp.float32),
          pltpu.VMEM((EPW,), jnp.int32),
      ],
  )(src)


NB_W = EPW // EB


EBW = 400
NB_W2 = EPW // EBW


def _w_body(src_hbm, dst_hbm, av_hbm, nm_hbm, out_hbm,
            av_v, nm_v, sidx0_v, didx0_v, sidx1_v, didx1_v,
            wout0_v, wout1_v, osem0, osem1):
  c = lax.axis_index("c")
  s = lax.axis_index("s")
  w = s * NC + c

  pltpu.sync_copy(av_hbm, av_v)
  pltpu.sync_copy(nm_hbm, nm_v)
  base = w * EPW
  iota = lax.iota(jnp.int32, L)

  def stage(i, sidx_v, didx_v):
    off = base + i * EBW
    pltpu.sync_copy(src_hbm.at[pl.ds(off, EBW)], sidx_v)
    pltpu.sync_copy(dst_hbm.at[pl.ds(off, EBW)], didx_v)

  def work(i, sidx_v, didx_v, wout_v, osem):
    @plsc.parallel_loop(0, EBW // L, 1, unroll=2)
    def group(g):
      sv = sidx_v[pl.ds(g * L, L)]
      dv = didx_v[pl.ds(g * L, L)]
      m = plsc.load_gather(nm_v, [sv]) * plsc.load_gather(nm_v, [dv])
      sv8 = sv * 8
      dv8 = dv * 8
      eid4 = (iota + g * L) * 4
      for h in range(HEADS):
        e = (plsc.load_gather(av_v, [sv8 + h]) +
             plsc.load_gather(av_v, [dv8 + (HEADS + h)]))
        e = jnp.where(e > 0, e, 0.2 * e)
        plsc.store_scatter(wout_v, [eid4 + h], jnp.exp(e) * m)
    pltpu.async_copy(wout_v, out_hbm.at[pl.ds((base + i * EBW) * 4, EBW * 4)],
                     osem)

  stage(0, sidx0_v, didx0_v)

  def pair(k, carry):
    a = 2 * k
    stage(a + 1, sidx1_v, didx1_v)

    @pl.when(k > 0)
    def _():
      pltpu.make_async_copy(
          wout0_v, out_hbm.at[pl.ds(base * 4, EBW * 4)], osem0).wait()
    work(a, sidx0_v, didx0_v, wout0_v, osem0)

    @pl.when(a + 2 < NB_W2)
    def _():
      stage(a + 2, sidx0_v, didx0_v)

    @pl.when(k > 0)
    def _():
      pltpu.make_async_copy(
          wout1_v, out_hbm.at[pl.ds(base * 4, EBW * 4)], osem1).wait()
    work(a + 1, sidx1_v, didx1_v, wout1_v, osem1)
    return carry
  lax.fori_loop(0, NB_W2 // 2, pair, 0)

  if NB_W2 % 2 == 1:
    pltpu.make_async_copy(
        wout0_v, out_hbm.at[pl.ds(base * 4, EBW * 4)], osem0).wait()
    work(NB_W2 - 1, sidx0_v, didx0_v, wout0_v, osem0)
  pltpu.make_async_copy(wout0_v, out_hbm.at[pl.ds(base * 4, EBW * 4)],
                        osem0).wait()
  pltpu.make_async_copy(wout1_v, out_hbm.at[pl.ds(base * 4, EBW * 4)],
                        osem1).wait()


def _w_call(src, dst, av, nm):
  return pl.kernel(
      _w_body,
      out_type=jax.ShapeDtypeStruct((E * 4,), jnp.float32),
      mesh=_sc_mesh(),
      compiler_params=_SC_PARAMS,
      scratch_types=[
          pltpu.VMEM((N * 8,), jnp.float32),
          pltpu.VMEM((N,), jnp.float32),
          pltpu.VMEM((EBW,), jnp.int32),
          pltpu.VMEM((EBW,), jnp.int32),
          pltpu.VMEM((EBW,), jnp.int32),
          pltpu.VMEM((EBW,), jnp.int32),
          pltpu.VMEM((EBW * 4,), jnp.float32),
          pltpu.VMEM((EBW * 4,), jnp.float32),
          pltpu.SemaphoreType.DMA,
          pltpu.SemaphoreType.DMA,
      ],
  )(src, dst, av, nm)


EPT2 = E // NW
NB_D = EPT2 // EB


def _den_body(dst_hbm, w_hbm, out_hbm,
              dbig_v, wbig_v, didx0s_v, didx1s_v,
              orow0_v, orow1_v, acc_sh, ssem0, ssem1):
  c = lax.axis_index("c")
  s = lax.axis_index("s")

  def zorow(r, carry):
    orow0_v[r, pl.ds(0, L)] = jnp.zeros((L,), jnp.float32)
    orow1_v[r, pl.ds(0, L)] = jnp.zeros((L,), jnp.float32)
    return carry
  lax.fori_loop(0, EB, zorow, 0)

  def zacc(i, carry):
    pltpu.sync_copy(orow0_v.at[pl.ds(0, 64)],
                    acc_sh.at[pl.ds(s * ROWS_PT + i * 64, 64)])
    return carry
  lax.fori_loop(0, ROWS_PT // 64, zacc, 0)
  plsc.subcore_barrier()

  base = (c * NS + s) * EPT2
  iota = lax.iota(jnp.int32, L)
  zi = jnp.zeros((L,), jnp.int32)
  NBD = EPT2 // SB

  def work(first, u, didxs_v, orow_v, ssem):
    for g in range(EB // L):
      eid = iota + g * L
      e4 = (eid + u * EB) * 4
      for j in range(HEADS):
        plsc.store_scatter(orow_v, [eid, zi + j],
                           plsc.load_gather(wbig_v, [e4 + j]))
      didxs_v[pl.ds(g * L, L)] = dbig_v[pl.ds(u * EB + g * L, L)]
    pltpu.async_copy(orow_v, acc_sh.at[didxs_v], ssem, add=True)

  def big(i, carry):
    off = base + i * SB
    pltpu.sync_copy(dst_hbm.at[pl.ds(off, SB)], dbig_v)
    pltpu.sync_copy(w_hbm.at[pl.ds(off * 4, SB * 4)], wbig_v)
    first = i == 0
    for u in range(SB // EB):
      even = u % 2 == 0
      didxsc, orowc, ssemc = (
          (didx0s_v, orow0_v, ssem0) if even else
          (didx1s_v, orow1_v, ssem1))

      @pl.when(jnp.logical_not(first) | (u >= 2))
      def _():
        pltpu.make_async_copy(orowc, acc_sh.at[didxsc], ssemc).wait()
      work(first, u, didxsc, orowc, ssemc)
    return carry
  lax.fori_loop(0, NBD, big, 0)

  pltpu.make_async_copy(orow0_v, acc_sh.at[didx0s_v], ssem0).wait()
  pltpu.make_async_copy(orow1_v, acc_sh.at[didx1s_v], ssem1).wait()
  plsc.subcore_barrier()
  pltpu.sync_copy(acc_sh.at[pl.ds(s * ROWS_PT, ROWS_PT)],
                  out_hbm.at[pl.ds(c * NP + s * ROWS_PT, ROWS_PT)])


def _den_call(dst, w):
  return pl.kernel(
      _den_body,
      out_type=jax.ShapeDtypeStruct((NC * NP, DW), jnp.float32),
      mesh=_sc_mesh(),
      compiler_params=_SC_PARAMS,
      scratch_types=[
          pltpu.VMEM((SB,), jnp.int32),
          pltpu.VMEM((SB * 4,), jnp.float32),
          pltpu.VMEM((EB,), jnp.int32),
          pltpu.VMEM((EB,), jnp.int32),
          pltpu.VMEM((EB, DW), jnp.float32),
          pltpu.VMEM((EB, DW), jnp.float32),
          pltpu.VMEM_SHARED((NP, DW), jnp.float32),
          pltpu.SemaphoreType.DMA,
          pltpu.SemaphoreType.DMA,
      ],
  )(dst, w)


SB = 400
NBB = EPT // SB


def _num_body(src_hbm, dst_hbm, xls_hbm, w_hbm, out_hbm,
              grows0_v, grows1_v, orow0_v, orow1_v,
              sbig_v, dbig_v, wbig_v,
              didx0s_v, didx1s_v, sidx20_v, sidx21_v,
              acc_sh, sem0, sem1, ssem0, ssem1):
  c = lax.axis_index("c")
  s = lax.axis_index("s")

  def zorow(r, carry):
    for j in range(2 * HID // L):
      orow0_v[r, pl.ds(j * L, L)] = jnp.zeros((L,), jnp.float32)
    return carry
  lax.fori_loop(0, EB, zorow, 0)

  def zacc(i, carry):
    pltpu.sync_copy(orow0_v.at[pl.ds(0, 64)],
                    acc_sh.at[pl.ds(s * ROWS_PT + i * 64, 64)])
    return carry
  lax.fori_loop(0, ROWS_PT // 64, zacc, 0)
  plsc.subcore_barrier()

  base = s * EPT
  h0 = c * 2

  def gstart(u, sidx2_v, grows_v, sem):
    for g in range(EB // L):
      sidx2_v[pl.ds(g * L, L)] = (
          sbig_v[pl.ds(u * EB + g * L, L)] + c * N)
    pltpu.async_copy(xls_hbm.at[sidx2_v], grows_v, sem)

  iota = lax.iota(jnp.int32, L)

  def work(first, u, didxs_v, grows_v, orow_v, ssem):
    for g in range(EB // L):
      e4 = (iota + u * EB + g * L) * 4 + h0
      wg0 = plsc.load_gather(wbig_v, [e4])
      wg1 = plsc.load_gather(wbig_v, [e4 + 1])

      @plsc.parallel_loop(0, L, 1, unroll=4)
      def scale(el):
        e = g * L + el
        lane = jnp.zeros((L,), jnp.int32) + el
        w0 = jnp.take(wg0, lane)
        w1 = jnp.take(wg1, lane)
        for f in range(HID // L):
          orow_v[e, pl.ds(f * L, L)] = grows_v[e, pl.ds(f * L, L)] * w0
          orow_v[e, pl.ds(HID + f * L, L)] = (
              grows_v[e, pl.ds(HID + f * L, L)] * w1)
    for g in range(EB // L):
      didxs_v[pl.ds(g * L, L)] = dbig_v[pl.ds(u * EB + g * L, L)]
    pltpu.async_copy(orow_v, acc_sh.at[didxs_v], ssem, add=True)

  def big(i, carry):
    off = base + i * SB
    pltpu.sync_copy(src_hbm.at[pl.ds(off, SB)], sbig_v)
    pltpu.sync_copy(dst_hbm.at[pl.ds(off, SB)], dbig_v)
    pltpu.sync_copy(w_hbm.at[pl.ds(off * 4, SB * 4)], wbig_v)
    first = i == 0
    gstart(0, sidx20_v, grows0_v, sem0)
    for u in range(SB // EB):
      even = u % 2 == 0
      sidx2n = sidx21_v if even else sidx20_v
      growsn = grows1_v if even else grows0_v
      semn = sem1 if even else sem0
      sidx2c, growsc, semc = (
          (sidx20_v, grows0_v, sem0) if even else
          (sidx21_v, grows1_v, sem1))
      didxsc, orowc, ssemc = (
          (didx0s_v, orow0_v, ssem0) if even else
          (didx1s_v, orow1_v, ssem1))
      if u + 1 < SB // EB:
        gstart(u + 1, sidx2n, growsn, semn)
      pltpu.make_async_copy(xls_hbm.at[sidx2c], growsc, semc).wait()

      @pl.when(jnp.logical_not(first) | (u >= 2))
      def _():
        pltpu.make_async_copy(orowc, acc_sh.at[didxsc], ssemc).wait()
      work(first, u, didxsc, growsc, orowc, ssemc)
    return carry
  lax.fori_loop(0, NBB, big, 0)

  pltpu.make_async_copy(orow0_v, acc_sh.at[didx0s_v], ssem0).wait()
  pltpu.make_async_copy(orow1_v, acc_sh.at[didx1s_v], ssem1).wait()
  plsc.subcore_barrier()
  pltpu.sync_copy(acc_sh.at[pl.ds(s * ROWS_PT, ROWS_PT)],
                  out_hbm.at[pl.ds(c * NP + s * ROWS_PT, ROWS_PT)])


def _num_call(src, dst, xls, w):
  return pl.kernel(
      _num_body,
      out_type=jax.ShapeDtypeStruct((NC * NP, 2 * HID), jnp.float32),
      mesh=_sc_mesh(),
      compiler_params=_SC_PARAMS,
      scratch_types=[
          pltpu.VMEM((EB, 2 * HID), jnp.float32),
          pltpu.VMEM((EB, 2 * HID), jnp.float32),
          pltpu.VMEM((EB, 2 * HID), jnp.float32),
          pltpu.VMEM((EB, 2 * HID), jnp.float32),
          pltpu.VMEM((SB,), jnp.int32),
          pltpu.VMEM((SB,), jnp.int32),
          pltpu.VMEM((SB * 4,), jnp.float32),
          pltpu.VMEM((EB,), jnp.int32),
          pltpu.VMEM((EB,), jnp.int32),
          pltpu.VMEM((EB,), jnp.int32),
          pltpu.VMEM((EB,), jnp.int32),
          pltpu.VMEM_SHARED((NP, 2 * HID), jnp.float32),
          pltpu.SemaphoreType.DMA,
          pltpu.SemaphoreType.DMA,
          pltpu.SemaphoreType.DMA,
          pltpu.SemaphoreType.DMA,
      ],
  )(src, dst, xls, w)


def _dense1_body(x_ref, w_ref, av_w_ref, xls_ref, av_ref):
  xl = jnp.dot(x_ref[...], w_ref[...], preferred_element_type=jnp.float32)
  xls_ref[0] = xl[:, :2 * HID]
  xls_ref[1] = xl[:, 2 * HID:]
  av_ref[...] = jnp.dot(xl, av_w_ref[...], preferred_element_type=jnp.float32)


def _dense1(x, W1, AV1):
  return pl.pallas_call(
      _dense1_body,
      grid=(GRID,),
      in_specs=[
          pl.BlockSpec((BR, IN_CH), lambda i: (i, 0)),
          pl.BlockSpec((IN_CH, HEADS * HID), lambda i: (0, 0)),
          pl.BlockSpec((HEADS * HID, 2 * HEADS), lambda i: (0, 0)),
      ],
      out_specs=[
          pl.BlockSpec((NC, BR, 2 * HID), lambda i: (0, i, 0)),
          pl.BlockSpec((BR, 2 * HEADS), lambda i: (i, 0)),
      ],
      out_shape=[
          jax.ShapeDtypeStruct((NC, N, 2 * HID), jnp.float32),
          jax.ShapeDtypeStruct((N, 2 * HEADS), jnp.float32),
      ],
  )(x, W1, AV1)


def _nmcomb_body(nmp_ref, nm_ref):
  nm_ref[...] = (jnp.sum(nmp_ref[...], axis=0) > 0).astype(jnp.float32)


def _nmcomb(nmp):
  return pl.pallas_call(
      _nmcomb_body,
      out_shape=jax.ShapeDtypeStruct((N,), jnp.float32),
  )(nmp)


def _finish_conv(ns, dn, xls, av, b, pw):
  num = jnp.concatenate([ns[0], ns[1]], axis=1)
  den = dn[0, :, :HEADS] + dn[1, :, :HEADS]
  xl = jnp.concatenate([xls[0], xls[1]], axis=1)
  es = av[:, :HEADS] + av[:, HEADS:]
  es = jnp.where(es > 0, es, 0.2 * es)
  wself = jnp.exp(es)
  br = xl.shape[0]
  w256 = jnp.broadcast_to(wself[:, :, None], (br, HEADS, HID)).reshape(
      br, HEADS * HID)
  d256 = jnp.broadcast_to((den + wself)[:, :, None], (br, HEADS, HID)).reshape(
      br, HEADS * HID)
  h = (num + w256 * xl) / (d256 + 1e-16) + b
  return jnp.where(h >= 0, h, pw * h)


def _dense2_body(ns_ref, dn_ref, xls_ref, av_ref, b_ref, w_ref, av_w_ref,
                 pw_ref, xls2_ref, av2_ref):
  h = _finish_conv(ns_ref[...], dn_ref[...], xls_ref[...], av_ref[...],
                   b_ref[...], pw_ref[0])
  xl2 = jnp.dot(h, w_ref[...], preferred_element_type=jnp.float32)
  xls2_ref[0] = xl2[:, :2 * HID]
  xls2_ref[1] = xl2[:, 2 * HID:]
  av2_ref[...] = jnp.dot(xl2, av_w_ref[...],
                         preferred_element_type=jnp.float32)


def _dense2(ns, dn, xls, av, b1, W2, AV2, pw):
  return pl.pallas_call(
      _dense2_body,
      grid=(GRID,),
      in_specs=[
          pl.BlockSpec((NC, BR, 2 * HID), lambda i: (0, i, 0)),
          pl.BlockSpec((NC, BR, DW), lambda i: (0, i, 0)),
          pl.BlockSpec((NC, BR, 2 * HID), lambda i: (0, i, 0)),
          pl.BlockSpec((BR, 2 * HEADS), lambda i: (i, 0)),
          pl.BlockSpec((1, HEADS * HID), lambda i: (0, 0)),
          pl.BlockSpec((HEADS * HID, HEADS * HID), lambda i: (0, 0)),
          pl.BlockSpec((HEADS * HID, 2 * HEADS), lambda i: (0, 0)),
          pl.BlockSpec(memory_space=pltpu.SMEM),
      ],
      out_specs=[
          pl.BlockSpec((NC, BR, 2 * HID), lambda i: (0, i, 0)),
          pl.BlockSpec((BR, 2 * HEADS), lambda i: (i, 0)),
      ],
      out_shape=[
          jax.ShapeDtypeStruct((NC, N, 2 * HID), jnp.float32),
          jax.ShapeDtypeStruct((N, 2 * HEADS), jnp.float32),
      ],
  )(ns, dn, xls, av, b1, W2, AV2, pw)


def _dense3_body(ns_ref, dn_ref, xls_ref, av_ref, b_ref, lpw_ref, pw_ref,
                 lpb_ref, out_ref):
  h = _finish_conv(ns_ref[...], dn_ref[...], xls_ref[...], av_ref[...],
                   b_ref[...], pw_ref[0])
  out = jnp.dot(h, lpw_ref[...], preferred_element_type=jnp.float32)
  out_ref[...] = out + lpb_ref[0]


def _dense3(ns, dn, xls, av, b2, lp_W, pw, lp_b):
  return pl.pallas_call(
      _dense3_body,
      grid=(GRID,),
      in_specs=[
          pl.BlockSpec((NC, BR, 2 * HID), lambda i: (0, i, 0)),
          pl.BlockSpec((NC, BR, DW), lambda i: (0, i, 0)),
          pl.BlockSpec((NC, BR, 2 * HID), lambda i: (0, i, 0)),
          pl.BlockSpec((BR, 2 * HEADS), lambda i: (i, 0)),
          pl.BlockSpec((1, HEADS * HID), lambda i: (0, 0)),
          pl.BlockSpec((HEADS * HID, 1), lambda i: (0, 0)),
          pl.BlockSpec(memory_space=pltpu.SMEM),
          pl.BlockSpec(memory_space=pltpu.SMEM),
      ],
      out_specs=pl.BlockSpec((BR, 1), lambda i: (i, 0)),
      out_shape=jax.ShapeDtypeStruct((N, 1), jnp.float32),
  )(ns, dn, xls, av, b2, lp_W, pw, lp_b)


def _att_matrix(att_src, att_dst):
  a = jnp.zeros((HEADS * HID, 2 * HEADS), jnp.float32)
  for h in range(HEADS):
    a = a.at[h * HID:(h + 1) * HID, h].set(att_src[h])
    a = a.at[h * HID:(h + 1) * HID, HEADS + h].set(att_dst[h])
  return a


@jax.jit
def kernel(x, edge_index, W1, att_src1, att_dst1, b1, W2, att_src2, att_dst2,
           b2, prelu_w, lp_W, lp_b):
  src = edge_index[0]
  dst = edge_index[1]
  AV1 = _att_matrix(att_src1, att_dst1)
  AV2 = _att_matrix(att_src2, att_dst2)
  pw = prelu_w.reshape(1)

  nm = _nmcomb(_nm_call(src).reshape(NW, N))
  xls1, av1 = _dense1(x, W1, AV1)
  wa1 = _w_call(src, dst, av1.reshape(N * 2 * HEADS), nm)
  dn1 = _den_call(dst, wa1)
  ns1 = _num_call(src, dst, xls1.reshape(NC * N, 2 * HID), wa1)
  xls2, av2 = _dense2(ns1.reshape(NC, NP, 2 * HID), dn1.reshape(NC, NP, DW),
                      xls1, av1, b1.reshape(1, -1), W2, AV2, pw)
  wa2 = _w_call(src, dst, av2.reshape(N * 2 * HEADS), nm)
  dn2 = _den_call(dst, wa2)
  ns2 = _num_call(src, dst, xls2.reshape(NC * N, 2 * HID), wa2)
  return _dense3(ns2.reshape(NC, NP, 2 * HID), dn2.reshape(NC, NP, DW),
                 xls2, av2, b2.reshape(1, -1), lp_W, pw, lp_b)[:, 0]


# speedup vs baseline: 1.2278x; 1.2278x over previous
import jax
import jax.numpy as jnp
from jax import lax
from jax.experimental import pallas as pl
from jax.experimental.pallas import tpu as pltpu
from jax.experimental.pallas import tpu_sc as plsc

N = 10000
E = 320000
IN_CH = 128
HID = 64
HEADS = 4

NC = 2
NS = 16
NW = NC * NS
L = 16

EB = 80
EPT = E // NS
NBATCH = EPT // EB
EPW = E // NW
NP = 10240
ROWS_PT = NP // NS
ZR = 32
DW = 16

BR = 200
GRID = N // BR

_SC_PARAMS = pltpu.CompilerParams(needs_layout_passes=False)


def _sc_mesh():
  return plsc.VectorSubcoreMesh(core_axis_name="c", subcore_axis_name="s")


def _nm_body(src_hbm, out_hbm, nm_v, idx_v):
  c = lax.axis_index("c")
  s = lax.axis_index("s")
  w = s * NC + c
  ones = jnp.ones((L,), jnp.float32)

  def zero(i, carry):
    nm_v[pl.ds(i * L, L)] = jnp.zeros((L,), jnp.float32)
    return carry
  lax.fori_loop(0, N // L, zero, 0)

  pltpu.sync_copy(src_hbm.at[pl.ds(w * EPW, EPW)], idx_v)

  def scat(i, carry):
    iv = idx_v[pl.ds(i * L, L)]
    plsc.store_scatter(nm_v, [iv], ones)
    return carry
  lax.fori_loop(0, EPW // L, scat, 0)

  pltpu.sync_copy(nm_v, out_hbm.at[pl.ds(w * N, N)])


def _nm_call(src):
  return pl.kernel(
      _nm_body,
      out_type=jax.ShapeDtypeStruct((NW * N,), jnp.float32),
      mesh=_sc_mesh(),
      compiler_params=_SC_PARAMS,
      scratch_types=[
          pltpu.VMEM((N,), jnp.float32),
          pltpu.VMEM((EPW,), jnp.int32),
      ],
  )(src)


NB_W = EPW // EB


EBW = 400
NB_W2 = EPW // EBW


def _w_body(src_hbm, dst_hbm, av_hbm, nm_hbm, out_hbm,
            av_v, nm_v, sidx0_v, didx0_v, sidx1_v, didx1_v,
            wout0_v, wout1_v, osem0, osem1):
  c = lax.axis_index("c")
  s = lax.axis_index("s")
  w = s * NC + c

  pltpu.sync_copy(av_hbm, av_v)
  pltpu.sync_copy(nm_hbm, nm_v)
  base = w * EPW
  iota = lax.iota(jnp.int32, L)

  def stage(i, sidx_v, didx_v):
    off = base + i * EBW
    pltpu.sync_copy(src_hbm.at[pl.ds(off, EBW)], sidx_v)
    pltpu.sync_copy(dst_hbm.at[pl.ds(off, EBW)], didx_v)

  def work(i, sidx_v, didx_v, wout_v, osem):
    @plsc.parallel_loop(0, EBW // L, 1, unroll=2)
    def group(g):
      sv = sidx_v[pl.ds(g * L, L)]
      dv = didx_v[pl.ds(g * L, L)]
      m = plsc.load_gather(nm_v, [sv]) * plsc.load_gather(nm_v, [dv])
      sv8 = sv * 8
      dv8 = dv * 8
      eid4 = (iota + g * L) * 4
      for h in range(HEADS):
        e = (plsc.load_gather(av_v, [sv8 + h]) +
             plsc.load_gather(av_v, [dv8 + (HEADS + h)]))
        e = jnp.where(e > 0, e, 0.2 * e)
        plsc.store_scatter(wout_v, [eid4 + h], jnp.exp(e) * m)
    pltpu.async_copy(wout_v, out_hbm.at[pl.ds((base + i * EBW) * 4, EBW * 4)],
                     osem)

  stage(0, sidx0_v, didx0_v)

  def pair(k, carry):
    a = 2 * k
    stage(a + 1, sidx1_v, didx1_v)

    @pl.when(k > 0)
    def _():
      pltpu.make_async_copy(
          wout0_v, out_hbm.at[pl.ds(base * 4, EBW * 4)], osem0).wait()
    work(a, sidx0_v, didx0_v, wout0_v, osem0)

    @pl.when(a + 2 < NB_W2)
    def _():
      stage(a + 2, sidx0_v, didx0_v)

    @pl.when(k > 0)
    def _():
      pltpu.make_async_copy(
          wout1_v, out_hbm.at[pl.ds(base * 4, EBW * 4)], osem1).wait()
    work(a + 1, sidx1_v, didx1_v, wout1_v, osem1)
    return carry
  lax.fori_loop(0, NB_W2 // 2, pair, 0)

  if NB_W2 % 2 == 1:
    pltpu.make_async_copy(
        wout0_v, out_hbm.at[pl.ds(base * 4, EBW * 4)], osem0).wait()
    work(NB_W2 - 1, sidx0_v, didx0_v, wout0_v, osem0)
  pltpu.make_async_copy(wout0_v, out_hbm.at[pl.ds(base * 4, EBW * 4)],
                        osem0).wait()
  pltpu.make_async_copy(wout1_v, out_hbm.at[pl.ds(base * 4, EBW * 4)],
                        osem1).wait()


def _w_call(src, dst, av, nm):
  return pl.kernel(
      _w_body,
      out_type=jax.ShapeDtypeStruct((E * 4,), jnp.float32),
      mesh=_sc_mesh(),
      compiler_params=_SC_PARAMS,
      scratch_types=[
          pltpu.VMEM((N * 8,), jnp.float32),
          pltpu.VMEM((N,), jnp.float32),
          pltpu.VMEM((EBW,), jnp.int32),
          pltpu.VMEM((EBW,), jnp.int32),
          pltpu.VMEM((EBW,), jnp.int32),
          pltpu.VMEM((EBW,), jnp.int32),
          pltpu.VMEM((EBW * 4,), jnp.float32),
          pltpu.VMEM((EBW * 4,), jnp.float32),
          pltpu.SemaphoreType.DMA,
          pltpu.SemaphoreType.DMA,
      ],
  )(src, dst, av, nm)


EPT2 = E // NW
NB_D = EPT2 // EB


def _den_body(dst_hbm, w_hbm, out_hbm,
              dbig_v, wbig_v, didx0s_v, didx1s_v,
              orow0_v, orow1_v, acc_sh, ssem0, ssem1):
  c = lax.axis_index("c")
  s = lax.axis_index("s")

  def zorow(r, carry):
    orow0_v[r, pl.ds(0, L)] = jnp.zeros((L,), jnp.float32)
    orow1_v[r, pl.ds(0, L)] = jnp.zeros((L,), jnp.float32)
    return carry
  lax.fori_loop(0, EB, zorow, 0)

  def zacc(i, carry):
    pltpu.sync_copy(orow0_v.at[pl.ds(0, 64)],
                    acc_sh.at[pl.ds(s * ROWS_PT + i * 64, 64)])
    return carry
  lax.fori_loop(0, ROWS_PT // 64, zacc, 0)
  plsc.subcore_barrier()

  base = (c * NS + s) * EPT2
  iota = lax.iota(jnp.int32, L)
  zi = jnp.zeros((L,), jnp.int32)
  NBD = EPT2 // SB

  def work(first, u, didxs_v, orow_v, ssem):
    for g in range(EB // L):
      eid = iota + g * L
      e4 = (eid + u * EB) * 4
      for j in range(HEADS):
        plsc.store_scatter(orow_v, [eid, zi + j],
                           plsc.load_gather(wbig_v, [e4 + j]))
      didxs_v[pl.ds(g * L, L)] = dbig_v[pl.ds(u * EB + g * L, L)]
    pltpu.async_copy(orow_v, acc_sh.at[didxs_v], ssem, add=True)

  def big(i, carry):
    off = base + i * SB
    pltpu.sync_copy(dst_hbm.at[pl.ds(off, SB)], dbig_v)
    pltpu.sync_copy(w_hbm.at[pl.ds(off * 4, SB * 4)], wbig_v)
    first = i == 0
    for u in range(SB // EB):
      even = u % 2 == 0
      didxsc, orowc, ssemc = (
          (didx0s_v, orow0_v, ssem0) if even else
          (didx1s_v, orow1_v, ssem1))

      @pl.when(jnp.logical_not(first) | (u >= 2))
      def _():
        pltpu.make_async_copy(orowc, acc_sh.at[didxsc], ssemc).wait()
      work(first, u, didxsc, orowc, ssemc)
    return carry
  lax.fori_loop(0, NBD, big, 0)

  pltpu.make_async_copy(orow0_v, acc_sh.at[didx0s_v], ssem0).wait()
  pltpu.make_async_copy(orow1_v, acc_sh.at[didx1s_v], ssem1).wait()
  plsc.subcore_barrier()
  pltpu.sync_copy(acc_sh.at[pl.ds(s * ROWS_PT, ROWS_PT)],
                  out_hbm.at[pl.ds(c * NP + s * ROWS_PT, ROWS_PT)])


def _den_call(dst, w):
  return pl.kernel(
      _den_body,
      out_type=jax.ShapeDtypeStruct((NC * NP, DW), jnp.float32),
      mesh=_sc_mesh(),
      compiler_params=_SC_PARAMS,
      scratch_types=[
          pltpu.VMEM((SB,), jnp.int32),
          pltpu.VMEM((SB * 4,), jnp.float32),
          pltpu.VMEM((EB,), jnp.int32),
          pltpu.VMEM((EB,), jnp.int32),
          pltpu.VMEM((EB, DW), jnp.float32),
          pltpu.VMEM((EB, DW), jnp.float32),
          pltpu.VMEM_SHARED((NP, DW), jnp.float32),
          pltpu.SemaphoreType.DMA,
          pltpu.SemaphoreType.DMA,
      ],
  )(dst, w)


SB = 400
NBB = EPT // SB


def _num_body(src_hbm, dst_hbm, xls_hbm, w_hbm, out_hbm,
              grows0_v, grows1_v, orow0_v, orow1_v,
              sbig_v, dbig_v, wbig_v,
              didx0s_v, didx1s_v, sidx20_v, sidx21_v,
              acc_sh, sem0, sem1, ssem0, ssem1):
  c = lax.axis_index("c")
  s = lax.axis_index("s")

  def zorow(r, carry):
    for j in range(2 * HID // L):
      orow0_v[r, pl.ds(j * L, L)] = jnp.zeros((L,), jnp.float32)
    return carry
  lax.fori_loop(0, EB, zorow, 0)

  def zacc(i, carry):
    pltpu.sync_copy(orow0_v.at[pl.ds(0, 64)],
                    acc_sh.at[pl.ds(s * ROWS_PT + i * 64, 64)])
    return carry
  lax.fori_loop(0, ROWS_PT // 64, zacc, 0)
  plsc.subcore_barrier()

  base = s * EPT
  h0 = c * 2

  def gstart(u, sidx2_v, grows_v, sem):
    for g in range(EB // L):
      sidx2_v[pl.ds(g * L, L)] = (
          sbig_v[pl.ds(u * EB + g * L, L)] + c * N)
    pltpu.async_copy(xls_hbm.at[sidx2_v], grows_v, sem)

  def work(first, u, didxs_v, grows_v, orow_v, ssem):
    @plsc.parallel_loop(0, EB, 1, unroll=4)
    def scale(e):
      ev4 = jnp.zeros((L,), jnp.int32) + (u * EB + e) * 4 + h0
      w0 = plsc.load_gather(wbig_v, [ev4])
      w1 = plsc.load_gather(wbig_v, [ev4 + 1])
      for f in range(HID // L):
        orow_v[e, pl.ds(f * L, L)] = grows_v[e, pl.ds(f * L, L)] * w0
        orow_v[e, pl.ds(HID + f * L, L)] = (
            grows_v[e, pl.ds(HID + f * L, L)] * w1)
    for g in range(EB // L):
      didxs_v[pl.ds(g * L, L)] = dbig_v[pl.ds(u * EB + g * L, L)]
    pltpu.async_copy(orow_v, acc_sh.at[didxs_v], ssem, add=True)

  def big(i, carry):
    off = base + i * SB
    pltpu.sync_copy(src_hbm.at[pl.ds(off, SB)], sbig_v)
    pltpu.sync_copy(dst_hbm.at[pl.ds(off, SB)], dbig_v)
    pltpu.sync_copy(w_hbm.at[pl.ds(off * 4, SB * 4)], wbig_v)
    first = i == 0
    gstart(0, sidx20_v, grows0_v, sem0)
    for u in range(SB // EB):
      even = u % 2 == 0
      sidx2n = sidx21_v if even else sidx20_v
      growsn = grows1_v if even else grows0_v
      semn = sem1 if even else sem0
      sidx2c, growsc, semc = (
          (sidx20_v, grows0_v, sem0) if even else
          (sidx21_v, grows1_v, sem1))
      didxsc, orowc, ssemc = (
          (didx0s_v, orow0_v, ssem0) if even else
          (didx1s_v, orow1_v, ssem1))
      if u + 1 < SB // EB:
        gstart(u + 1, sidx2n, growsn, semn)
      pltpu.make_async_copy(xls_hbm.at[sidx2c], growsc, semc).wait()

      @pl.when(jnp.logical_not(first) | (u >= 2))
      def _():
        pltpu.make_async_copy(orowc, acc_sh.at[didxsc], ssemc).wait()
      work(first, u, didxsc, growsc, orowc, ssemc)
    return carry
  lax.fori_loop(0, NBB, big, 0)

  pltpu.make_async_copy(orow0_v, acc_sh.at[didx0s_v], ssem0).wait()
  pltpu.make_async_copy(orow1_v, acc_sh.at[didx1s_v], ssem1).wait()
  plsc.subcore_barrier()
  pltpu.sync_copy(acc_sh.at[pl.ds(s * ROWS_PT, ROWS_PT)],
                  out_hbm.at[pl.ds(c * NP + s * ROWS_PT, ROWS_PT)])


def _num_call(src, dst, xls, w):
  return pl.kernel(
      _num_body,
      out_type=jax.ShapeDtypeStruct((NC * NP, 2 * HID), jnp.float32),
      mesh=_sc_mesh(),
      compiler_params=_SC_PARAMS,
      scratch_types=[
          pltpu.VMEM((EB, 2 * HID), jnp.float32),
          pltpu.VMEM((EB, 2 * HID), jnp.float32),
          pltpu.VMEM((EB, 2 * HID), jnp.float32),
          pltpu.VMEM((EB, 2 * HID), jnp.float32),
          pltpu.VMEM((SB,), jnp.int32),
          pltpu.VMEM((SB,), jnp.int32),
          pltpu.VMEM((SB * 4,), jnp.float32),
          pltpu.VMEM((EB,), jnp.int32),
          pltpu.VMEM((EB,), jnp.int32),
          pltpu.VMEM((EB,), jnp.int32),
          pltpu.VMEM((EB,), jnp.int32),
          pltpu.VMEM_SHARED((NP, 2 * HID), jnp.float32),
          pltpu.SemaphoreType.DMA,
          pltpu.SemaphoreType.DMA,
          pltpu.SemaphoreType.DMA,
          pltpu.SemaphoreType.DMA,
      ],
  )(src, dst, xls, w)


def _dense1_body(x_ref, w_ref, av_w_ref, xls_ref, av_ref):
  xl = jnp.dot(x_ref[...], w_ref[...], preferred_element_type=jnp.float32)
  xls_ref[0] = xl[:, :2 * HID]
  xls_ref[1] = xl[:, 2 * HID:]
  av_ref[...] = jnp.dot(xl, av_w_ref[...], preferred_element_type=jnp.float32)


def _dense1(x, W1, AV1):
  return pl.pallas_call(
      _dense1_body,
      grid=(GRID,),
      in_specs=[
          pl.BlockSpec((BR, IN_CH), lambda i: (i, 0)),
          pl.BlockSpec((IN_CH, HEADS * HID), lambda i: (0, 0)),
          pl.BlockSpec((HEADS * HID, 2 * HEADS), lambda i: (0, 0)),
      ],
      out_specs=[
          pl.BlockSpec((NC, BR, 2 * HID), lambda i: (0, i, 0)),
          pl.BlockSpec((BR, 2 * HEADS), lambda i: (i, 0)),
      ],
      out_shape=[
          jax.ShapeDtypeStruct((NC, N, 2 * HID), jnp.float32),
          jax.ShapeDtypeStruct((N, 2 * HEADS), jnp.float32),
      ],
  )(x, W1, AV1)


def _nmcomb_body(nmp_ref, nm_ref):
  nm_ref[...] = (jnp.sum(nmp_ref[...], axis=0) > 0).astype(jnp.float32)


def _nmcomb(nmp):
  return pl.pallas_call(
      _nmcomb_body,
      out_shape=jax.ShapeDtypeStruct((N,), jnp.float32),
  )(nmp)


def _finish_conv(ns, dn, xls, av, b, pw):
  num = jnp.concatenate([ns[0], ns[1]], axis=1)
  den = dn[0, :, :HEADS] + dn[1, :, :HEADS]
  xl = jnp.concatenate([xls[0], xls[1]], axis=1)
  es = av[:, :HEADS] + av[:, HEADS:]
  es = jnp.where(es > 0, es, 0.2 * es)
  wself = jnp.exp(es)
  br = xl.shape[0]
  w256 = jnp.broadcast_to(wself[:, :, None], (br, HEADS, HID)).reshape(
      br, HEADS * HID)
  d256 = jnp.broadcast_to((den + wself)[:, :, None], (br, HEADS, HID)).reshape(
      br, HEADS * HID)
  h = (num + w256 * xl) / (d256 + 1e-16) + b
  return jnp.where(h >= 0, h, pw * h)


def _dense2_body(ns_ref, dn_ref, xls_ref, av_ref, b_ref, w_ref, av_w_ref,
                 pw_ref, xls2_ref, av2_ref):
  h = _finish_conv(ns_ref[...], dn_ref[...], xls_ref[...], av_ref[...],
                   b_ref[...], pw_ref[0])
  xl2 = jnp.dot(h, w_ref[...], preferred_element_type=jnp.float32)
  xls2_ref[0] = xl2[:, :2 * HID]
  xls2_ref[1] = xl2[:, 2 * HID:]
  av2_ref[...] = jnp.dot(xl2, av_w_ref[...],
                         preferred_element_type=jnp.float32)


def _dense2(ns, dn, xls, av, b1, W2, AV2, pw):
  return pl.pallas_call(
      _dense2_body,
      grid=(GRID,),
      in_specs=[
          pl.BlockSpec((NC, BR, 2 * HID), lambda i: (0, i, 0)),
          pl.BlockSpec((NC, BR, DW), lambda i: (0, i, 0)),
          pl.BlockSpec((NC, BR, 2 * HID), lambda i: (0, i, 0)),
          pl.BlockSpec((BR, 2 * HEADS), lambda i: (i, 0)),
          pl.BlockSpec((1, HEADS * HID), lambda i: (0, 0)),
          pl.BlockSpec((HEADS * HID, HEADS * HID), lambda i: (0, 0)),
          pl.BlockSpec((HEADS * HID, 2 * HEADS), lambda i: (0, 0)),
          pl.BlockSpec(memory_space=pltpu.SMEM),
      ],
      out_specs=[
          pl.BlockSpec((NC, BR, 2 * HID), lambda i: (0, i, 0)),
          pl.BlockSpec((BR, 2 * HEADS), lambda i: (i, 0)),
      ],
      out_shape=[
          jax.ShapeDtypeStruct((NC, N, 2 * HID), jnp.float32),
          jax.ShapeDtypeStruct((N, 2 * HEADS), jnp.float32),
      ],
  )(ns, dn, xls, av, b1, W2, AV2, pw)


def _dense3_body(ns_ref, dn_ref, xls_ref, av_ref, b_ref, lpw_ref, pw_ref,
                 lpb_ref, out_ref):
  h = _finish_conv(ns_ref[...], dn_ref[...], xls_ref[...], av_ref[...],
                   b_ref[...], pw_ref[0])
  out = jnp.dot(h, lpw_ref[...], preferred_element_type=jnp.float32)
  out_ref[...] = out + lpb_ref[0]


def _dense3(ns, dn, xls, av, b2, lp_W, pw, lp_b):
  return pl.pallas_call(
      _dense3_body,
      grid=(GRID,),
      in_specs=[
          pl.BlockSpec((NC, BR, 2 * HID), lambda i: (0, i, 0)),
          pl.BlockSpec((NC, BR, DW), lambda i: (0, i, 0)),
          pl.BlockSpec((NC, BR, 2 * HID), lambda i: (0, i, 0)),
          pl.BlockSpec((BR, 2 * HEADS), lambda i: (i, 0)),
          pl.BlockSpec((1, HEADS * HID), lambda i: (0, 0)),
          pl.BlockSpec((HEADS * HID, 1), lambda i: (0, 0)),
          pl.BlockSpec(memory_space=pltpu.SMEM),
          pl.BlockSpec(memory_space=pltpu.SMEM),
      ],
      out_specs=pl.BlockSpec((BR, 1), lambda i: (i, 0)),
      out_shape=jax.ShapeDtypeStruct((N, 1), jnp.float32),
  )(ns, dn, xls, av, b2, lp_W, pw, lp_b)


def _att_matrix(att_src, att_dst):
  a = jnp.zeros((HEADS * HID, 2 * HEADS), jnp.float32)
  for h in range(HEADS):
    a = a.at[h * HID:(h + 1) * HID, h].set(att_src[h])
    a = a.at[h * HID:(h + 1) * HID, HEADS + h].set(att_dst[h])
  return a


@jax.jit
def kernel(x, edge_index, W1, att_src1, att_dst1, b1, W2, att_src2, att_dst2,
           b2, prelu_w, lp_W, lp_b):
  src = edge_index[0]
  dst = edge_index[1]
  AV1 = _att_matrix(att_src1, att_dst1)
  AV2 = _att_matrix(att_src2, att_dst2)
  pw = prelu_w.reshape(1)

  nm = _nmcomb(_nm_call(src).reshape(NW, N))
  xls1, av1 = _dense1(x, W1, AV1)
  wa1 = _w_call(src, dst, av1.reshape(N * 2 * HEADS), nm)
  dn1 = _den_call(dst, wa1)
  ns1 = _num_call(src, dst, xls1.reshape(NC * N, 2 * HID), wa1)
  xls2, av2 = _dense2(ns1.reshape(NC, NP, 2 * HID), dn1.reshape(NC, NP, DW),
                      xls1, av1, b1.reshape(1, -1), W2, AV2, pw)
  wa2 = _w_call(src, dst, av2.reshape(N * 2 * HEADS), nm)
  dn2 = _den_call(dst, wa2)
  ns2 = _num_call(src, dst, xls2.reshape(NC * N, 2 * HID), wa2)
  return _dense3(ns2.reshape(NC, NP, 2 * HID), dn2.reshape(NC, NP, DW),
                 xls2, av2, b2.reshape(1, -1), lp_W, pw, lp_b)[:, 0]
